# baseline (speedup 1.0000x reference)
import functools

import jax
import jax.numpy as jnp
from jax.experimental import pallas as pl

B, N, M, Q, C, S, NC = 2, 10000, 2048, 128, 256, 2, 20
TN = 1024
NP = 10240
H = 128
CS = C // S

_HI = jax.lax.Precision.HIGHEST


def _bf(x):
    return x.astype(jnp.bfloat16).astype(jnp.float32)


def _dot16(x, w):
    return jnp.dot(x.astype(jnp.bfloat16), w.astype(jnp.bfloat16),
                   preferred_element_type=jnp.float32)


def _cls(x, w1t, b1, g1, be1, w2t, b2):
    h = _dot16(x, w1t) + b1
    h = g1 * h + be1
    h = jax.nn.gelu(h)
    return _dot16(h, w2t) + b2


def _mlp(x, w1t, b1, g1, be1, w2t, b2, g2, be2):
    h = _dot16(x, w1t) + b1
    h = jax.nn.gelu(g1 * h + be1)
    h = _dot16(h, w2t) + b2
    return jax.nn.gelu(g2 * h + be2)


def _head_kernel(xpf_ref, xqf_ref, qt_ref, orig_ref,
                 wpt_ref, wqt_ref, wot_ref,
                 qw1t_ref, qb1_ref, qg1_ref, qbe1_ref, qw2t_ref, qb2_ref,
                 pw1t_ref, pb1_ref, pg1_ref, pbe1_ref, pw2t_ref, pb2_ref,
                 pft_ref, ppred_ref, qpred_ref, qidx_ref):
    xp = xpf_ref[0]
    xq = xqf_ref[0]

    pf_lin = jnp.dot(xp, wpt_ref[...])
    qf_lin = jnp.dot(xq, wqt_ref[...])
    logits = jax.lax.dot_general(
        pf_lin, qf_lin, (((1,), (1,)), ((), ()))) / jnp.sqrt(jnp.float32(C))
    attn = jax.nn.softmax(logits, axis=-1)
    agg = jnp.dot(attn, xq)
    pft = xp + jnp.dot(agg, wot_ref[...])
    pft_ref[...] = pft[None]

    ppred_ref[...] = _cls(pft, pw1t_ref[...], pb1_ref[...], pg1_ref[...],
                          pbe1_ref[...], pw2t_ref[...], pb2_ref[...])[None]
    qpred_ref[...] = _cls(xq, qw1t_ref[...], qb1_ref[...], qg1_ref[...],
                          qbe1_ref[...], qw2t_ref[...], qb2_ref[...])[None]

    qt = qt_ref[0]
    og = orig_ref[0]
    aa = (qt[:, 0:1] * qt[:, 0:1] + qt[:, 1:2] * qt[:, 1:2]
          + qt[:, 2:3] * qt[:, 2:3])
    o0, o1, o2 = og[0:1, :], og[1:2, :], og[2:3, :]
    bb = o0 * o0 + o1 * o1 + o2 * o2
    ab = (_bf(qt[:, 0:1]) * _bf(o0) + _bf(qt[:, 1:2]) * _bf(o1)
          + _bf(qt[:, 2:3]) * _bf(o2))
    dq = aa + bb - 2.0 * ab
    mn = jnp.min(dq, axis=1, keepdims=True)
    col = jax.lax.broadcasted_iota(jnp.int32, dq.shape, 1)
    qidx_ref[...] = jnp.min(jnp.where(dq == mn, col, N), axis=1,
                            keepdims=True)[None]


def _main_kernel(unk_ref, pts_ref, pft_ref, l01_ref,
                 m0w1t_ref, m0b1_ref, m0g1_ref, m0be1_ref,
                 m0w2t_ref, m0b2_ref, m0g2_ref, m0be2_ref,
                 m1w1t_ref, m1b1_ref, m1g1_ref, m1be1_ref,
                 m1w2t_ref, m1b2_ref, m1g2_ref, m1be2_ref,
                 cw1t_ref, cb1_ref, cg1_ref, cbe1_ref, cw2t_ref, cb2_ref,
                 preds_ref, logp_ref):
    u = unk_ref[0]
    k = pts_ref[0]

    u0, u1, u2 = u[:, 0:1], u[:, 1:2], u[:, 2:3]
    k0, k1, k2 = k[0:1, :], k[1:2, :], k[2:3, :]
    aa = u0 * u0 + u1 * u1 + u2 * u2
    bb = k0 * k0 + k1 * k1 + k2 * k2
    ab = _bf(u0) * _bf(k0) + _bf(u1) * _bf(k1) + _bf(u2) * _bf(k2)
    d2 = aa + bb - 2.0 * ab

    col = jax.lax.broadcasted_iota(jnp.int32, d2.shape, 1)
    inf = jnp.float32(jnp.inf)
    d = d2
    dists, idxs = [], []
    for _ in range(3):
        mn = jnp.min(d, axis=1, keepdims=True)
        am = jnp.min(jnp.where(d == mn, col, M), axis=1, keepdims=True)
        dists.append(mn)
        idxs.append(am)
        d = jnp.where(col == am, inf, d)

    r0 = 1.0 / (dists[0] + 1e-8)
    r1 = 1.0 / (dists[1] + 1e-8)
    r2 = 1.0 / (dists[2] + 1e-8)
    norm = r0 + r1 + r2
    w0, w1, w2 = r0 / norm, r1 / norm, r2 / norm

    zero = jnp.float32(0.0)
    a = (jnp.where(col == idxs[0], w0, zero)
         + jnp.where(col == idxs[1], w1, zero)
         + jnp.where(col == idxs[2], w2, zero))

    interp = jnp.dot(a, pft_ref[0], precision=_HI)
    g01 = jnp.dot(a, l01_ref[0], precision=_HI)

    interp = interp + _mlp(g01[:, CS:], m1w1t_ref[...], m1b1_ref[...],
                           m1g1_ref[...], m1be1_ref[...], m1w2t_ref[...],
                           m1b2_ref[...], m1g2_ref[...], m1be2_ref[...])
    interp = interp + _mlp(g01[:, :CS], m0w1t_ref[...], m0b1_ref[...],
                           m0g1_ref[...], m0be1_ref[...], m0w2t_ref[...],
                           m0b2_ref[...], m0g2_ref[...], m0be2_ref[...])

    preds = _cls(interp, cw1t_ref[...], cb1_ref[...], cg1_ref[...],
                 cbe1_ref[...], cw2t_ref[...], cb2_ref[...])
    preds_ref[...] = preds[None]
    logp_ref[...] = jax.nn.log_softmax(preds, axis=-1)[None]


def _full(shape):
    n = len(shape)
    return pl.BlockSpec(shape, lambda *_: (0,) * n)


def _row2(v):
    return v.reshape(1, -1)


@jax.jit
def kernel(original_points, points, query, query_feats, points_feats,
           points_feats_list_0, points_feats_list_1, params, points_index):
    del points_index
    f32 = jnp.float32

    xpf = points_feats.transpose(0, 2, 1)
    xqf = query_feats.transpose(0, 2, 1)
    qt = query.transpose(0, 2, 1)

    qfp = params['qfp']
    qc, pc, cc = (params['query_classifier'], params['points_classifier'],
                  params['classifier'])

    def cls_args(p):
        return [p['W1'].T, _row2(p['b1']), _row2(p['g1']), _row2(p['be1']),
                p['W2'].T, _row2(p['b2'])]

    head_in = ([xpf, xqf, qt, original_points,
                qfp['Wp'].T, qfp['Wq'].T, qfp['Wo'].T]
               + cls_args(qc) + cls_args(pc))
    head_specs = ([pl.BlockSpec((1, M, C), lambda b: (b, 0, 0)),
                   pl.BlockSpec((1, Q, C), lambda b: (b, 0, 0)),
                   pl.BlockSpec((1, Q, 3), lambda b: (b, 0, 0)),
                   pl.BlockSpec((1, 3, N), lambda b: (b, 0, 0))]
                  + [_full(x.shape) for x in head_in[4:]])

    pft, ppred, qpred, qidx = pl.pallas_call(
        _head_kernel,
        grid=(B,),
        in_specs=head_specs,
        out_specs=[pl.BlockSpec((1, M, C), lambda b: (b, 0, 0)),
                   pl.BlockSpec((1, M, NC), lambda b: (b, 0, 0)),
                   pl.BlockSpec((1, Q, NC), lambda b: (b, 0, 0)),
                   pl.BlockSpec((1, Q, 1), lambda b: (b, 0, 0))],
        out_shape=[jax.ShapeDtypeStruct((B, M, C), f32),
                   jax.ShapeDtypeStruct((B, M, NC), f32),
                   jax.ShapeDtypeStruct((B, Q, NC), f32),
                   jax.ShapeDtypeStruct((B, Q, 1), jnp.int32)],
    )(*head_in)

    unk = original_points.transpose(0, 2, 1)
    unk = jnp.pad(unk, ((0, 0), (0, NP - N), (0, 0)))
    l01 = jnp.concatenate([points_feats_list_0.transpose(0, 2, 1),
                           points_feats_list_1.transpose(0, 2, 1)], axis=2)

    def mlp_args(p):
        return [p['W1'].T, _row2(p['b1']), _row2(p['g1']), _row2(p['be1']),
                p['W2'].T, _row2(p['b2']), _row2(p['g2']), _row2(p['be2'])]

    main_in = ([unk, points, pft, l01]
               + mlp_args(params['mlp'][0]) + mlp_args(params['mlp'][1])
               + cls_args(cc))
    main_specs = ([pl.BlockSpec((1, TN, 3), lambda b, t: (b, t, 0)),
                   pl.BlockSpec((1, 3, M), lambda b, t: (b, 0, 0)),
                   pl.BlockSpec((1, M, C), lambda b, t: (b, 0, 0)),
                   pl.BlockSpec((1, M, C), lambda b, t: (b, 0, 0))]
                  + [_full(x.shape) for x in main_in[4:]])

    preds_p, logp_p = pl.pallas_call(
        _main_kernel,
        grid=(B, NP // TN),
        in_specs=main_specs,
        out_specs=[pl.BlockSpec((1, TN, NC), lambda b, t: (b, t, 0)),
                   pl.BlockSpec((1, TN, NC), lambda b, t: (b, t, 0))],
        out_shape=[jax.ShapeDtypeStruct((B, NP, NC), f32),
                   jax.ShapeDtypeStruct((B, NP, NC), f32)],
    )(*main_in)

    return (qpred, qidx.reshape(B, Q), ppred,
            preds_p[:, :N, :], logp_p[:, :N, :])


# speedup vs baseline: 1.0216x; 1.0216x over previous
import functools

import jax
import jax.numpy as jnp
from jax.experimental import pallas as pl

B, N, M, Q, C, S, NC = 2, 10000, 2048, 128, 256, 2, 20
TN = 512
NP = 10240
H = 128
CS = C // S

_HI = jax.lax.Precision.HIGHEST


def _bf(x):
    return x.astype(jnp.bfloat16).astype(jnp.float32)


def _dot16(x, w):
    return jnp.dot(x.astype(jnp.bfloat16), w.astype(jnp.bfloat16),
                   preferred_element_type=jnp.float32)


def _cls(x, w1t, b1, g1, be1, w2t, b2):
    h = _dot16(x, w1t) + b1
    h = g1 * h + be1
    h = jax.nn.gelu(h)
    return _dot16(h, w2t) + b2


def _mlp(x, w1t, b1, g1, be1, w2t, b2, g2, be2):
    h = _dot16(x, w1t) + b1
    h = jax.nn.gelu(g1 * h + be1)
    h = _dot16(h, w2t) + b2
    return jax.nn.gelu(g2 * h + be2)


def _head_kernel(xpf_ref, xqf_ref, qt_ref, orig_ref,
                 wpt_ref, wqt_ref, wot_ref,
                 qw1t_ref, qb1_ref, qg1_ref, qbe1_ref, qw2t_ref, qb2_ref,
                 pw1t_ref, pb1_ref, pg1_ref, pbe1_ref, pw2t_ref, pb2_ref,
                 pft_ref, ppred_ref, qpred_ref, qidx_ref):
    xp = xpf_ref[0]
    xq = xqf_ref[0]

    pf_lin = jnp.dot(xp, wpt_ref[...])
    qf_lin = jnp.dot(xq, wqt_ref[...])
    logits = jax.lax.dot_general(
        pf_lin, qf_lin, (((1,), (1,)), ((), ()))) / jnp.sqrt(jnp.float32(C))
    attn = jax.nn.softmax(logits, axis=-1)
    agg = jnp.dot(attn, xq)
    pft = xp + jnp.dot(agg, wot_ref[...])
    pft_ref[...] = pft[None]

    ppred_ref[...] = _cls(pft, pw1t_ref[...], pb1_ref[...], pg1_ref[...],
                          pbe1_ref[...], pw2t_ref[...], pb2_ref[...])[None]
    qpred_ref[...] = _cls(xq, qw1t_ref[...], qb1_ref[...], qg1_ref[...],
                          qbe1_ref[...], qw2t_ref[...], qb2_ref[...])[None]

    qt = qt_ref[0]
    og = orig_ref[0]
    aa = (qt[:, 0:1] * qt[:, 0:1] + qt[:, 1:2] * qt[:, 1:2]
          + qt[:, 2:3] * qt[:, 2:3])
    o0, o1, o2 = og[0:1, :], og[1:2, :], og[2:3, :]
    bb = o0 * o0 + o1 * o1 + o2 * o2
    ab = (_bf(qt[:, 0:1]) * _bf(o0) + _bf(qt[:, 1:2]) * _bf(o1)
          + _bf(qt[:, 2:3]) * _bf(o2))
    dq = aa + bb - 2.0 * ab
    mn = jnp.min(dq, axis=1, keepdims=True)
    col = jax.lax.broadcasted_iota(jnp.int32, dq.shape, 1)
    qidx_ref[...] = jnp.min(jnp.where(dq == mn, col, N), axis=1,
                            keepdims=True)[None]


def _main_kernel(unk_ref, pts_ref, pft_ref, l01_ref,
                 m0w1t_ref, m0b1_ref, m0g1_ref, m0be1_ref,
                 m0w2t_ref, m0b2_ref, m0g2_ref, m0be2_ref,
                 m1w1t_ref, m1b1_ref, m1g1_ref, m1be1_ref,
                 m1w2t_ref, m1b2_ref, m1g2_ref, m1be2_ref,
                 cw1t_ref, cb1_ref, cg1_ref, cbe1_ref, cw2t_ref, cb2_ref,
                 preds_ref, logp_ref):
    u = unk_ref[0]
    k = pts_ref[0]

    u0, u1, u2 = u[:, 0:1], u[:, 1:2], u[:, 2:3]
    k0, k1, k2 = k[0:1, :], k[1:2, :], k[2:3, :]
    aa = u0 * u0 + u1 * u1 + u2 * u2
    bb = k0 * k0 + k1 * k1 + k2 * k2
    ab = _bf(u0) * _bf(k0) + _bf(u1) * _bf(k1) + _bf(u2) * _bf(k2)
    d2 = aa + bb - 2.0 * ab

    col = jax.lax.broadcasted_iota(jnp.int32, d2.shape, 1)
    inf = jnp.float32(jnp.inf)
    d = d2
    dists, idxs = [], []
    for _ in range(3):
        mn = jnp.min(d, axis=1, keepdims=True)
        am = jnp.min(jnp.where(d == mn, col, M), axis=1, keepdims=True)
        dists.append(mn)
        idxs.append(am)
        d = jnp.where(col == am, inf, d)

    r0 = 1.0 / (dists[0] + 1e-8)
    r1 = 1.0 / (dists[1] + 1e-8)
    r2 = 1.0 / (dists[2] + 1e-8)
    norm = r0 + r1 + r2
    w0, w1, w2 = r0 / norm, r1 / norm, r2 / norm

    zero = jnp.float32(0.0)
    a = (jnp.where(col == idxs[0], w0, zero)
         + jnp.where(col == idxs[1], w1, zero)
         + jnp.where(col == idxs[2], w2, zero))

    interp = jnp.dot(a, pft_ref[0], precision=_HI)
    g01 = jnp.dot(a, l01_ref[0], precision=_HI)

    interp = interp + _mlp(g01[:, CS:], m1w1t_ref[...], m1b1_ref[...],
                           m1g1_ref[...], m1be1_ref[...], m1w2t_ref[...],
                           m1b2_ref[...], m1g2_ref[...], m1be2_ref[...])
    interp = interp + _mlp(g01[:, :CS], m0w1t_ref[...], m0b1_ref[...],
                           m0g1_ref[...], m0be1_ref[...], m0w2t_ref[...],
                           m0b2_ref[...], m0g2_ref[...], m0be2_ref[...])

    preds = _cls(interp, cw1t_ref[...], cb1_ref[...], cg1_ref[...],
                 cbe1_ref[...], cw2t_ref[...], cb2_ref[...])
    preds_ref[...] = preds[None]
    logp_ref[...] = jax.nn.log_softmax(preds, axis=-1)[None]


def _full(shape):
    n = len(shape)
    return pl.BlockSpec(shape, lambda *_: (0,) * n)


def _row2(v):
    return v.reshape(1, -1)


@jax.jit
def kernel(original_points, points, query, query_feats, points_feats,
           points_feats_list_0, points_feats_list_1, params, points_index):
    del points_index
    f32 = jnp.float32

    xpf = points_feats.transpose(0, 2, 1)
    xqf = query_feats.transpose(0, 2, 1)
    qt = query.transpose(0, 2, 1)

    qfp = params['qfp']
    qc, pc, cc = (params['query_classifier'], params['points_classifier'],
                  params['classifier'])

    def cls_args(p):
        return [p['W1'].T, _row2(p['b1']), _row2(p['g1']), _row2(p['be1']),
                p['W2'].T, _row2(p['b2'])]

    head_in = ([xpf, xqf, qt, original_points,
                qfp['Wp'].T, qfp['Wq'].T, qfp['Wo'].T]
               + cls_args(qc) + cls_args(pc))
    head_specs = ([pl.BlockSpec((1, M, C), lambda b: (b, 0, 0)),
                   pl.BlockSpec((1, Q, C), lambda b: (b, 0, 0)),
                   pl.BlockSpec((1, Q, 3), lambda b: (b, 0, 0)),
                   pl.BlockSpec((1, 3, N), lambda b: (b, 0, 0))]
                  + [_full(x.shape) for x in head_in[4:]])

    pft, ppred, qpred, qidx = pl.pallas_call(
        _head_kernel,
        grid=(B,),
        in_specs=head_specs,
        out_specs=[pl.BlockSpec((1, M, C), lambda b: (b, 0, 0)),
                   pl.BlockSpec((1, M, NC), lambda b: (b, 0, 0)),
                   pl.BlockSpec((1, Q, NC), lambda b: (b, 0, 0)),
                   pl.BlockSpec((1, Q, 1), lambda b: (b, 0, 0))],
        out_shape=[jax.ShapeDtypeStruct((B, M, C), f32),
                   jax.ShapeDtypeStruct((B, M, NC), f32),
                   jax.ShapeDtypeStruct((B, Q, NC), f32),
                   jax.ShapeDtypeStruct((B, Q, 1), jnp.int32)],
    )(*head_in)

    unk = original_points.transpose(0, 2, 1)
    unk = jnp.pad(unk, ((0, 0), (0, NP - N), (0, 0)))
    l01 = jnp.concatenate([points_feats_list_0.transpose(0, 2, 1),
                           points_feats_list_1.transpose(0, 2, 1)], axis=2)

    def mlp_args(p):
        return [p['W1'].T, _row2(p['b1']), _row2(p['g1']), _row2(p['be1']),
                p['W2'].T, _row2(p['b2']), _row2(p['g2']), _row2(p['be2'])]

    main_in = ([unk, points, pft, l01]
               + mlp_args(params['mlp'][0]) + mlp_args(params['mlp'][1])
               + cls_args(cc))
    main_specs = ([pl.BlockSpec((1, TN, 3), lambda b, t: (b, t, 0)),
                   pl.BlockSpec((1, 3, M), lambda b, t: (b, 0, 0)),
                   pl.BlockSpec((1, M, C), lambda b, t: (b, 0, 0)),
                   pl.BlockSpec((1, M, C), lambda b, t: (b, 0, 0))]
                  + [_full(x.shape) for x in main_in[4:]])

    preds_p, logp_p = pl.pallas_call(
        _main_kernel,
        grid=(B, NP // TN),
        in_specs=main_specs,
        out_specs=[pl.BlockSpec((1, TN, NC), lambda b, t: (b, t, 0)),
                   pl.BlockSpec((1, TN, NC), lambda b, t: (b, t, 0))],
        out_shape=[jax.ShapeDtypeStruct((B, NP, NC), f32),
                   jax.ShapeDtypeStruct((B, NP, NC), f32)],
    )(*main_in)

    return (qpred, qidx.reshape(B, Q), ppred,
            preds_p[:, :N, :], logp_p[:, :N, :])


# speedup vs baseline: 2.1001x; 2.0558x over previous
import jax
import jax.numpy as jnp
from jax.experimental import pallas as pl

B, N, M, Q, C, S, NC = 2, 10000, 2048, 128, 256, 2, 20
TN = 512
NP = 10240
H = 128
CS = C // S

BF = jnp.bfloat16


def _dotp(x, y):
    return jnp.dot(x, y, preferred_element_type=jnp.float32)


def _dot16(x, w):
    return _dotp(x.astype(BF), w.astype(BF))


def _hilo(x):
    h = x.astype(BF)
    return h, (x - h.astype(jnp.float32)).astype(BF)


def _cls(x, w1t, b1, g1, be1, w2t, b2):
    h = _dot16(x, w1t) + b1
    h = g1 * h + be1
    h = jax.nn.gelu(h)
    return _dot16(h, w2t) + b2


def _mlp(x, w1t, b1, g1, be1, w2t, b2, g2, be2):
    h = _dot16(x, w1t) + b1
    h = jax.nn.gelu(g1 * h + be1)
    h = _dot16(h, w2t) + b2
    return jax.nn.gelu(g2 * h + be2)


def _head_kernel(xpf_ref, xqf_ref, qt_ref, orig_ref,
                 wpt_ref, wqt_ref, wot_ref,
                 qw1t_ref, qb1_ref, qg1_ref, qbe1_ref, qw2t_ref, qb2_ref,
                 pw1t_ref, pb1_ref, pg1_ref, pbe1_ref, pw2t_ref, pb2_ref,
                 pfh_ref, pfl_ref, ppred_ref, qpred_ref, qidx_ref):
    xp = xpf_ref[0]
    xq = xqf_ref[0]

    pf_lin = jnp.dot(xp, wpt_ref[...])
    qf_lin = jnp.dot(xq, wqt_ref[...])
    logits = jax.lax.dot_general(
        pf_lin, qf_lin, (((1,), (1,)), ((), ()))) / jnp.sqrt(jnp.float32(C))
    attn = jax.nn.softmax(logits, axis=-1)
    agg = jnp.dot(attn, xq)
    pft = xp + jnp.dot(agg, wot_ref[...])
    pfh, pfl = _hilo(pft)
    pfh_ref[...] = pfh[None]
    pfl_ref[...] = pfl[None]

    ppred_ref[...] = _cls(pft, pw1t_ref[...], pb1_ref[...], pg1_ref[...],
                          pbe1_ref[...], pw2t_ref[...], pb2_ref[...])[None]
    qpred_ref[...] = _cls(xq, qw1t_ref[...], qb1_ref[...], qg1_ref[...],
                          qbe1_ref[...], qw2t_ref[...], qb2_ref[...])[None]

    qt = qt_ref[0]
    og = orig_ref[0]
    aa = (qt[:, 0:1] * qt[:, 0:1] + qt[:, 1:2] * qt[:, 1:2]
          + qt[:, 2:3] * qt[:, 2:3])
    o0, o1, o2 = og[0:1, :], og[1:2, :], og[2:3, :]
    bb = o0 * o0 + o1 * o1 + o2 * o2
    ab = _dot16(qt, og)
    dq = aa + bb - 2.0 * ab
    mn = jnp.min(dq, axis=1, keepdims=True)
    col = jax.lax.broadcasted_iota(jnp.int32, dq.shape, 1)
    qidx_ref[...] = jnp.min(jnp.where(dq == mn, col, N), axis=1,
                            keepdims=True)[None]


def _main_kernel(unk_ref, pts_ref, pfh_ref, pfl_ref, l01_ref,
                 m0w1t_ref, m0b1_ref, m0g1_ref, m0be1_ref,
                 m0w2t_ref, m0b2_ref, m0g2_ref, m0be2_ref,
                 m1w1t_ref, m1b1_ref, m1g1_ref, m1be1_ref,
                 m1w2t_ref, m1b2_ref, m1g2_ref, m1be2_ref,
                 cw1t_ref, cb1_ref, cg1_ref, cbe1_ref, cw2t_ref, cb2_ref,
                 preds_ref, logp_ref):
    u = unk_ref[0]
    k = pts_ref[0]

    u0, u1, u2 = u[:, 0:1], u[:, 1:2], u[:, 2:3]
    k0, k1, k2 = k[0:1, :], k[1:2, :], k[2:3, :]
    aa = u0 * u0 + u1 * u1 + u2 * u2
    bb = k0 * k0 + k1 * k1 + k2 * k2
    ab = _dot16(u, k)
    d2 = aa + bb - 2.0 * ab

    col = jax.lax.broadcasted_iota(jnp.int32, d2.shape, 1)
    inf = jnp.float32(jnp.inf)
    d = d2
    dists, idxs = [], []
    for _ in range(3):
        mn = jnp.min(d, axis=1, keepdims=True)
        am = jnp.min(jnp.where(d == mn, col, M), axis=1, keepdims=True)
        dists.append(mn)
        idxs.append(am)
        d = jnp.where(col == am, inf, d)

    r0 = 1.0 / (dists[0] + 1e-8)
    r1 = 1.0 / (dists[1] + 1e-8)
    r2 = 1.0 / (dists[2] + 1e-8)
    norm = r0 + r1 + r2
    w0, w1, w2 = r0 / norm, r1 / norm, r2 / norm

    zero = jnp.float32(0.0)
    a = (jnp.where(col == idxs[0], w0, zero)
         + jnp.where(col == idxs[1], w1, zero)
         + jnp.where(col == idxs[2], w2, zero))
    ah, al = _hilo(a)

    interp = (_dotp(ah, pfh_ref[0]) + _dotp(ah, pfl_ref[0])
              + _dotp(al, pfh_ref[0]))
    g01 = _dotp(ah, l01_ref[0])

    interp = interp + _mlp(g01[:, CS:], m1w1t_ref[...], m1b1_ref[...],
                           m1g1_ref[...], m1be1_ref[...], m1w2t_ref[...],
                           m1b2_ref[...], m1g2_ref[...], m1be2_ref[...])
    interp = interp + _mlp(g01[:, :CS], m0w1t_ref[...], m0b1_ref[...],
                           m0g1_ref[...], m0be1_ref[...], m0w2t_ref[...],
                           m0b2_ref[...], m0g2_ref[...], m0be2_ref[...])

    preds = _cls(interp, cw1t_ref[...], cb1_ref[...], cg1_ref[...],
                 cbe1_ref[...], cw2t_ref[...], cb2_ref[...])
    preds_ref[...] = preds[None]
    logp_ref[...] = jax.nn.log_softmax(preds, axis=-1)[None]


def _full(shape):
    n = len(shape)
    return pl.BlockSpec(shape, lambda *_: (0,) * n)


def _row2(v):
    return v.reshape(1, -1)


@jax.jit
def kernel(original_points, points, query, query_feats, points_feats,
           points_feats_list_0, points_feats_list_1, params, points_index):
    del points_index
    f32 = jnp.float32

    xpf = points_feats.transpose(0, 2, 1)
    xqf = query_feats.transpose(0, 2, 1)
    qt = query.transpose(0, 2, 1)

    qfp = params['qfp']
    qc, pc, cc = (params['query_classifier'], params['points_classifier'],
                  params['classifier'])

    def cls_args(p):
        return [p['W1'].T.astype(BF), _row2(p['b1']), _row2(p['g1']),
                _row2(p['be1']), p['W2'].T.astype(BF), _row2(p['b2'])]

    head_in = ([xpf, xqf, qt, original_points,
                qfp['Wp'].T, qfp['Wq'].T, qfp['Wo'].T]
               + cls_args(qc) + cls_args(pc))
    head_specs = ([pl.BlockSpec((1, M, C), lambda b: (b, 0, 0)),
                   pl.BlockSpec((1, Q, C), lambda b: (b, 0, 0)),
                   pl.BlockSpec((1, Q, 3), lambda b: (b, 0, 0)),
                   pl.BlockSpec((1, 3, N), lambda b: (b, 0, 0))]
                  + [_full(x.shape) for x in head_in[4:]])

    pfh, pfl, ppred, qpred, qidx = pl.pallas_call(
        _head_kernel,
        grid=(B,),
        in_specs=head_specs,
        out_specs=[pl.BlockSpec((1, M, C), lambda b: (b, 0, 0)),
                   pl.BlockSpec((1, M, C), lambda b: (b, 0, 0)),
                   pl.BlockSpec((1, M, NC), lambda b: (b, 0, 0)),
                   pl.BlockSpec((1, Q, NC), lambda b: (b, 0, 0)),
                   pl.BlockSpec((1, Q, 1), lambda b: (b, 0, 0))],
        out_shape=[jax.ShapeDtypeStruct((B, M, C), BF),
                   jax.ShapeDtypeStruct((B, M, C), BF),
                   jax.ShapeDtypeStruct((B, M, NC), f32),
                   jax.ShapeDtypeStruct((B, Q, NC), f32),
                   jax.ShapeDtypeStruct((B, Q, 1), jnp.int32)],
    )(*head_in)

    unk = original_points.transpose(0, 2, 1)
    unk = jnp.pad(unk, ((0, 0), (0, NP - N), (0, 0)))
    l01 = jnp.concatenate(
        [points_feats_list_0.transpose(0, 2, 1),
         points_feats_list_1.transpose(0, 2, 1)], axis=2).astype(BF)

    def mlp_args(p):
        return [p['W1'].T.astype(BF), _row2(p['b1']), _row2(p['g1']),
                _row2(p['be1']), p['W2'].T.astype(BF), _row2(p['b2']),
                _row2(p['g2']), _row2(p['be2'])]

    main_in = ([unk, points, pfh, pfl, l01]
               + mlp_args(params['mlp'][0]) + mlp_args(params['mlp'][1])
               + cls_args(cc))
    main_specs = ([pl.BlockSpec((1, TN, 3), lambda b, t: (b, t, 0)),
                   pl.BlockSpec((1, 3, M), lambda b, t: (b, 0, 0)),
                   pl.BlockSpec((1, M, C), lambda b, t: (b, 0, 0)),
                   pl.BlockSpec((1, M, C), lambda b, t: (b, 0, 0)),
                   pl.BlockSpec((1, M, C), lambda b, t: (b, 0, 0))]
                  + [_full(x.shape) for x in main_in[5:]])

    preds_p, logp_p = pl.pallas_call(
        _main_kernel,
        grid=(B, NP // TN),
        in_specs=main_specs,
        out_specs=[pl.BlockSpec((1, TN, NC), lambda b, t: (b, t, 0)),
                   pl.BlockSpec((1, TN, NC), lambda b, t: (b, t, 0))],
        out_shape=[jax.ShapeDtypeStruct((B, NP, NC), f32),
                   jax.ShapeDtypeStruct((B, NP, NC), f32)],
    )(*main_in)

    return (qpred, qidx.reshape(B, Q), ppred,
            preds_p[:, :N, :], logp_p[:, :N, :])
